# baseline (speedup 1.0000x reference)
import functools

import jax
import jax.numpy as jnp
from jax.experimental import pallas as pl
from jax.experimental.pallas import tpu as pltpu


def _norm_select_kernel(x_ref, kx_ref, norms_ref, *, nb, nc, d, k, n):
    c = pl.program_id(1)
    xb = x_ref[0]
    sq = xb * xb
    n64 = sq[:, 0:d] + sq[:, d : 2 * d] + sq[:, 2 * d : 3 * d]
    u64 = jax.lax.bitcast_convert_type(n64, jnp.int32)
    half = nc // 2

    @pl.when(c < half)
    def _():
        norms_ref[pl.ds(c * nb, nb), 0:d] = u64

    @pl.when(c >= half)
    def _():
        norms_ref[pl.ds((c - half) * nb, nb), d : 2 * d] = u64

    @pl.when(c == nc - 1)
    def _():
        un = norms_ref[...]
        ones_row = jnp.ones((1, n // 2), jnp.float32)

        def body(_, carry):
            lo, hi = carry
            mid = jax.lax.shift_right_logical(lo + hi, 1)
            ind = jnp.where(un <= mid, 1.0, 0.0)
            cnt = jax.lax.dot_general(
                ones_row, ind, (((1,), (0,)), ((), ())),
                preferred_element_type=jnp.float32,
            )
            cnt = cnt + jnp.roll(cnt, d, axis=1)
            pred = cnt >= float(k)
            lo2 = jnp.where(pred, lo, mid + 1)
            hi2 = jnp.where(pred, mid, hi)
            return (lo2, hi2)

        lo0 = jnp.zeros((1, 2 * d), jnp.int32)
        hi0 = jnp.full((1, 2 * d), jnp.int32(0x7FFFFFFF))
        lo, _ = jax.lax.fori_loop(0, 31, body, (lo0, hi0))
        kx = jax.lax.bitcast_convert_type(lo, jnp.float32)
        kx_ref[0] = kx[:, 0:d]


def _mask_kernel(x_ref, kx_ref, o_ref, *, d):
    xb = x_ref[0]
    sq = xb * xb
    n64 = sq[:, 0:d] + sq[:, d : 2 * d] + sq[:, 2 * d : 3 * d]
    m = (n64 > kx_ref[0]).astype(jnp.float32)
    o_ref[0, :, 0, :] = xb[:, 0:d] * m
    o_ref[0, :, 1, :] = xb[:, d : 2 * d] * m
    o_ref[0, :, 2, :] = xb[:, 2 * d : 3 * d] * m


def kernel(x):
    b, n, c3, d = x.shape
    assert c3 == 3
    k = n // 10
    l3 = c3 * d

    nb_a = 2048
    nc_a = n // nb_a
    nb_b = 512
    nc_b = n // nb_b

    xr = x.reshape(b, n, l3)

    kx = pl.pallas_call(
        functools.partial(_norm_select_kernel, nb=nb_a, nc=nc_a, d=d, k=k, n=n),
        grid=(b, nc_a),
        in_specs=[pl.BlockSpec((1, nb_a, l3), lambda bi, ci: (bi, ci, 0))],
        out_specs=pl.BlockSpec((1, 1, d), lambda bi, ci: (bi, 0, 0)),
        out_shape=jax.ShapeDtypeStruct((b, 1, d), jnp.float32),
        scratch_shapes=[pltpu.VMEM((n // 2, 2 * d), jnp.int32)],
    )(xr)

    out = pl.pallas_call(
        functools.partial(_mask_kernel, d=d),
        grid=(b, nc_b),
        in_specs=[
            pl.BlockSpec((1, nb_b, l3), lambda bi, ci: (bi, ci, 0)),
            pl.BlockSpec((1, 1, d), lambda bi, ci: (bi, 0, 0)),
        ],
        out_specs=pl.BlockSpec((1, nb_b, 3, d), lambda bi, ci: (bi, ci, 0, 0)),
        out_shape=jax.ShapeDtypeStruct((b, n, c3, d), jnp.float32),
    )(xr, kx)

    return out


# speedup vs baseline: 1.6406x; 1.6406x over previous
import functools

import jax
import jax.numpy as jnp
from jax.experimental import pallas as pl
from jax.experimental.pallas import tpu as pltpu


def _norm_select_kernel(x_ref, kx_ref, norms_ref, *, nb, nc, d, k, n):
    c = pl.program_id(1)
    xb = x_ref[0]
    sq = xb * xb
    n64 = sq[:, 0:d] + sq[:, d : 2 * d] + sq[:, 2 * d : 3 * d]
    u64 = jax.lax.bitcast_convert_type(n64, jnp.int32)
    half = nc // 2

    @pl.when(c < half)
    def _():
        norms_ref[pl.ds(c * nb, nb), 0:d] = u64

    @pl.when(c >= half)
    def _():
        norms_ref[pl.ds((c - half) * nb, nb), d : 2 * d] = u64

    @pl.when(c == nc - 1)
    def _():
        un = norms_ref[...]
        ones_row = jnp.ones((1, n // 2), jnp.float32)

        def body(_, carry):
            lo, hi = carry
            mid = jax.lax.shift_right_logical(lo + hi, 1)
            ind = jnp.where(un <= mid, 1.0, 0.0)
            cnt = jax.lax.dot_general(
                ones_row, ind, (((1,), (0,)), ((), ())),
                preferred_element_type=jnp.float32,
            )
            cnt = cnt + jnp.roll(cnt, d, axis=1)
            pred = cnt >= float(k)
            lo2 = jnp.where(pred, lo, mid + 1)
            hi2 = jnp.where(pred, mid, hi)
            return (lo2, hi2)

        lo0 = jnp.zeros((1, 2 * d), jnp.int32)
        hi0 = jnp.full((1, 2 * d), jnp.int32(0x7FFFFFFF))
        lo, _ = jax.lax.fori_loop(0, 31, body, (lo0, hi0))
        kx = jax.lax.bitcast_convert_type(lo, jnp.float32)
        kx_ref[0] = kx[:, 0:d]


def _mask_kernel(x_ref, kx_ref, o_ref, *, d):
    xb = x_ref[0]
    sq = xb * xb
    n64 = sq[:, 0:d] + sq[:, d : 2 * d] + sq[:, 2 * d : 3 * d]
    m = (n64 > kx_ref[0]).astype(jnp.float32)
    m3 = jnp.concatenate([m, m, m], axis=-1)
    o_ref[0] = xb * m3


def kernel(x):
    b, n, c3, d = x.shape
    assert c3 == 3
    k = n // 10
    l3 = c3 * d

    nb_a = 2048
    nc_a = n // nb_a
    nb_b = 2048
    nc_b = n // nb_b

    xr = x.reshape(b, n, l3)

    kx = pl.pallas_call(
        functools.partial(_norm_select_kernel, nb=nb_a, nc=nc_a, d=d, k=k, n=n),
        grid=(b, nc_a),
        in_specs=[pl.BlockSpec((1, nb_a, l3), lambda bi, ci: (bi, ci, 0))],
        out_specs=pl.BlockSpec((1, 1, d), lambda bi, ci: (bi, 0, 0)),
        out_shape=jax.ShapeDtypeStruct((b, 1, d), jnp.float32),
        scratch_shapes=[pltpu.VMEM((n // 2, 2 * d), jnp.int32)],
    )(xr)

    out = pl.pallas_call(
        functools.partial(_mask_kernel, d=d),
        grid=(b, nc_b),
        in_specs=[
            pl.BlockSpec((1, nb_b, l3), lambda bi, ci: (bi, ci, 0)),
            pl.BlockSpec((1, 1, d), lambda bi, ci: (bi, 0, 0)),
        ],
        out_specs=pl.BlockSpec((1, nb_b, l3), lambda bi, ci: (bi, ci, 0)),
        out_shape=jax.ShapeDtypeStruct((b, n, l3), jnp.float32),
    )(xr, kx)

    return out.reshape(b, n, c3, d)
